# baseline (speedup 1.0000x reference)
import functools

import jax
import jax.numpy as jnp
from jax import lax
from jax.experimental import pallas as pl
from jax.experimental.pallas import tpu as pltpu
from jax.experimental.pallas import tpu_sc as plsc

_B = 4096
_EMB = 32
_HIST = 50
_NC = 2
_NS = 16
_NW = _NC * _NS
_BPW = _B // _NW
_HALF = _BPW // 2
_CPH = _HALF * _HIST // _BPW

_B4 = 8192
_BN = 4 * _B4


def _pack_transpose(tabT):
    V = tabT.shape[1]
    G = pl.cdiv(V, _BN)

    def body(x0, x1, x2, x3, o_ref):
        X = jnp.concatenate([x0[...], x1[...], x2[...], x3[...]], axis=0)
        rows = lax.broadcasted_iota(jnp.int32, (128, 128), 0)
        cols = lax.broadcasted_iota(jnp.int32, (128, 128), 1)
        R = ((rows // 32 == cols // 32) & (rows % 32 == cols % 32)
             ).astype(jnp.float32)
        o_ref[...] = jax.lax.dot_general(
            X, R, (((0,), (0,)), ((), ())),
            preferred_element_type=jnp.float32)

    out = pl.pallas_call(
        body,
        grid=(G,),
        in_specs=[pl.BlockSpec(
            (32, _B4),
            lambda i, k=k: (0, jnp.minimum(4 * i + k, pl.cdiv(V, _B4) - 1)))
                  for k in range(4)],
        out_specs=pl.BlockSpec((_B4, 128), lambda i: (i, 0)),
        out_shape=jax.ShapeDtypeStruct((G * _B4, 128), jnp.float32),
    )(tabT, tabT, tabT, tabT)
    return out.reshape(G * _B4 * 4, _EMB)


def _remap(v):
    i = v // _BN
    r = v % _BN
    return ((i * _B4 + r % _B4) * 4 + r // _B4).astype(jnp.int32)


def _sc_smalls(sp_i, it_i, ng_i, sp_tabT, item_tabT):
    mesh = plsc.VectorSubcoreMesh(core_axis_name="c", subcore_axis_name="s")
    out_t = [jax.ShapeDtypeStruct((_B, _EMB), jnp.float32)] * 3
    K = 16

    @functools.partial(
        pl.kernel,
        out_type=out_t,
        mesh=mesh,
        scratch_types=[
            pltpu.VMEM((_BPW,), jnp.int32),
            pltpu.VMEM((K, 32, 128), jnp.float32),
            pltpu.VMEM((_BPW, _EMB), jnp.float32),
            pltpu.SemaphoreType.DMA,
        ],
        compiler_params=pltpu.CompilerParams(needs_layout_passes=False),
    )
    def k(sp_i_h, it_i_h, ng_i_h, sp_tab_h, item_tab_h,
          sp_out, true_out, neg_out,
          ids, colbuf, ext_s, sem):
        wid = lax.axis_index("s") * _NC + lax.axis_index("c")
        iota = lax.broadcasted_iota(jnp.int32, (16,), 0)

        def feature(src_i_h, tab, out_h):
            pltpu.sync_copy(src_i_h.at[wid], ids)

            def vat(i):
                v16 = ids[pl.ds((i // 16) * 16, 16)]
                m = iota == (i % 16)
                sv = lax.reduce_max(jnp.where(m, v16, -1), (0,))
                col = pl.multiple_of((sv // 128) * 128, 128)
                lane = lax.gather(
                    v16 & 127, jnp.full((16, 1), 0, jnp.int32) + (i % 16),
                    lax.GatherDimensionNumbers(
                        offset_dims=(), collapsed_slice_dims=(0,),
                        start_index_map=(0,)),
                    (1,), mode=lax.GatherScatterMode.PROMISE_IN_BOUNDS)
                return col, lane

            def fire(i):
                col, _ = vat(i)
                pltpu.async_copy(tab.at[:, pl.ds(col, 128)],
                                 colbuf.at[i % K], sem)

            for i in range(K):
                fire(i)

            def item(i, _):
                pltpu.make_async_copy(
                    tab.at[:, pl.ds(0, 128)], colbuf.at[i % K], sem).wait()

                @pl.when(i + K < _BPW)
                def _():
                    fire(i + K)

                sv = jnp.full((16,), 0, jnp.int32) + (i % K)
                _, lane = vat(i)
                x0 = plsc.load_gather(colbuf, [sv, iota, lane])
                x1 = plsc.load_gather(colbuf, [sv, iota + 16, lane])
                ext_s[i, pl.ds(0, 16)] = x0
                ext_s[i, pl.ds(16, 16)] = x1
                return 0

            lax.fori_loop(0, _BPW, item, 0)
            pltpu.sync_copy(ext_s, out_h.at[pl.ds(wid * _BPW, _BPW), :])

        feature(sp_i_h, sp_tab_h, sp_out)
        feature(it_i_h, item_tab_h, true_out)
        feature(ng_i_h, item_tab_h, neg_out)

    return k(sp_i, it_i, ng_i, sp_tabT, item_tabT)


def _sc_hist(h_i, hist_tab):
    mesh = plsc.VectorSubcoreMesh(core_axis_name="c", subcore_axis_name="s")

    @functools.partial(
        pl.kernel,
        out_type=[jax.ShapeDtypeStruct((_B, _EMB), jnp.float32)],
        mesh=mesh,
        scratch_types=[
            pltpu.VMEM((_HIST, _BPW), jnp.int32),
            pltpu.VMEM((_HALF * _HIST, _EMB), jnp.float32),
            pltpu.VMEM((_BPW, _EMB), jnp.float32),
            pltpu.SemaphoreType.DMA,
        ],
        compiler_params=pltpu.CompilerParams(use_tc_tiling_on_sc=False, needs_layout_passes=False),
    )
    def k(h_i_h, hist_tab_h, hist_out, hidx, buf, pooled, sem):
        wid = lax.axis_index("s") * _NC + lax.axis_index("c")
        base = wid * _BPW

        pltpu.sync_copy(h_i_h.at[wid], hidx)

        inv = jnp.float32(1.0 / _HIST)
        for h in range(2):
            copies = [
                pltpu.async_copy(hist_tab_h.at[hidx.at[h * _CPH + c]],
                                 buf.at[pl.ds(c * _BPW, _BPW), :], sem)
                for c in range(_CPH)
            ]
            for cp in copies:
                cp.wait()

            def row_body(r, _):
                def inner(j, accs):
                    a0, a1 = accs
                    rb = r * _HIST + j * 5
                    for t in range(5):
                        a0 = a0 + buf[rb + t, pl.ds(0, 16)]
                        a1 = a1 + buf[rb + t, pl.ds(16, 16)]
                    return a0, a1

                z = jnp.zeros((16,), jnp.float32)
                a0, a1 = lax.fori_loop(0, _HIST // 5, inner, (z, z))
                rr = h * _HALF + r
                pooled[rr, pl.ds(0, 16)] = a0 * inv
                pooled[rr, pl.ds(16, 16)] = a1 * inv
                return 0

            lax.fori_loop(0, _HALF, row_body, 0)

        pltpu.sync_copy(pooled, hist_out.at[pl.ds(base, _BPW), :])

    return k(h_i, hist_tab)[0]


def _tc_loss(u_dense, sp_emb, hist_pooled, true_emb, neg_emb, W1, b1, W2, b2):
    def body(ud, sp, hp, te, ne, w1, b1r, w2, b2r, out):
        x = jnp.concatenate([ud[...], sp[...], hp[...]], axis=1)
        h = jnp.maximum(
            jnp.dot(x, w1[...], preferred_element_type=jnp.float32) + b1r[...], 0.0)
        ue = jnp.dot(h, w2[...], preferred_element_type=jnp.float32) + b2r[...]
        tl = jnp.sum(ue * te[...], axis=1, keepdims=True)
        nl = jnp.sum(ue * ne[...], axis=1, keepdims=True)
        d = nl - tl
        m = jnp.maximum(d, 0.0)
        out[...] = m + jnp.log(jnp.exp(-m) + jnp.exp(d - m))

    return pl.pallas_call(
        body,
        out_shape=jax.ShapeDtypeStruct((_B, 1), jnp.float32),
    )(u_dense, sp_emb, hist_pooled, true_emb, neg_emb,
      W1, b1.reshape(1, -1), W2, b2.reshape(1, -1))


def kernel(u_dense, u_sparse, u_hist, item_id, neg_ids,
           sparse_table, hist_table, item_table, W1, b1, W2, b2):
    sp_i = u_sparse.astype(jnp.int32).reshape(_NW, _BPW)
    it_i = item_id.astype(jnp.int32).reshape(_NW, _BPW)
    ng_i = neg_ids.astype(jnp.int32).reshape(_NW, _BPW)
    h_i = _remap(u_hist.astype(jnp.int32).reshape(_B * _HIST))
    h_i = h_i.reshape(_NW, _HIST, _BPW)
    sp_emb, true_emb, neg_emb = _sc_smalls(
        sp_i, it_i, ng_i, sparse_table.T, item_table.T)
    hist_tab = _pack_transpose(hist_table.T)
    h_i, _ = lax.optimization_barrier((h_i, sp_emb))
    hist_pooled = _sc_hist(h_i, hist_tab)
    loss = _tc_loss(u_dense, sp_emb, hist_pooled, true_emb, neg_emb, W1, b1, W2, b2)
    return loss.reshape(_B)


# speedup vs baseline: 1.0012x; 1.0012x over previous
import functools

import jax
import jax.numpy as jnp
from jax import lax
from jax.experimental import pallas as pl
from jax.experimental.pallas import tpu as pltpu
from jax.experimental.pallas import tpu_sc as plsc

_B = 4096
_EMB = 32
_HIST = 50
_NC = 2
_NS = 16
_NW = _NC * _NS
_BPW = _B // _NW
_HALF = _BPW // 2
_CPH = _HALF * _HIST // _BPW

_B4 = 4096
_BN = 4 * _B4


def _pack_transpose(tabT):
    V = tabT.shape[1]
    G = pl.cdiv(V, _BN)

    def body(x0, x1, x2, x3, o_ref):
        X = jnp.concatenate([x0[...], x1[...], x2[...], x3[...]], axis=0)
        rows = lax.broadcasted_iota(jnp.int32, (128, 128), 0)
        cols = lax.broadcasted_iota(jnp.int32, (128, 128), 1)
        R = ((rows // 32 == cols // 32) & (rows % 32 == cols % 32)
             ).astype(jnp.float32)
        o_ref[...] = jax.lax.dot_general(
            X, R, (((0,), (0,)), ((), ())),
            preferred_element_type=jnp.float32)

    out = pl.pallas_call(
        body,
        grid=(G,),
        in_specs=[pl.BlockSpec(
            (32, _B4),
            lambda i, k=k: (0, jnp.minimum(4 * i + k, pl.cdiv(V, _B4) - 1)))
                  for k in range(4)],
        out_specs=pl.BlockSpec((_B4, 128), lambda i: (i, 0)),
        out_shape=jax.ShapeDtypeStruct((G * _B4, 128), jnp.float32),
    )(tabT, tabT, tabT, tabT)
    return out.reshape(G * _B4 * 4, _EMB)


def _remap(v):
    i = v // _BN
    r = v % _BN
    return ((i * _B4 + r % _B4) * 4 + r // _B4).astype(jnp.int32)


def _sc_smalls(sp_i, it_i, ng_i, sp_tabT, item_tabT):
    mesh = plsc.VectorSubcoreMesh(core_axis_name="c", subcore_axis_name="s")
    out_t = [jax.ShapeDtypeStruct((_B, _EMB), jnp.float32)] * 3
    K = 16

    @functools.partial(
        pl.kernel,
        out_type=out_t,
        mesh=mesh,
        scratch_types=[
            pltpu.VMEM((_BPW,), jnp.int32),
            pltpu.VMEM((K, 32, 128), jnp.float32),
            pltpu.VMEM((_BPW, _EMB), jnp.float32),
            pltpu.SemaphoreType.DMA,
        ],
        compiler_params=pltpu.CompilerParams(needs_layout_passes=False),
    )
    def k(sp_i_h, it_i_h, ng_i_h, sp_tab_h, item_tab_h,
          sp_out, true_out, neg_out,
          ids, colbuf, ext_s, sem):
        wid = lax.axis_index("s") * _NC + lax.axis_index("c")
        iota = lax.broadcasted_iota(jnp.int32, (16,), 0)

        def feature(src_i_h, tab, out_h):
            pltpu.sync_copy(src_i_h.at[wid], ids)

            def vat(i):
                v16 = ids[pl.ds((i // 16) * 16, 16)]
                m = iota == (i % 16)
                sv = lax.reduce_max(jnp.where(m, v16, -1), (0,))
                col = pl.multiple_of((sv // 128) * 128, 128)
                lane = lax.gather(
                    v16 & 127, jnp.full((16, 1), 0, jnp.int32) + (i % 16),
                    lax.GatherDimensionNumbers(
                        offset_dims=(), collapsed_slice_dims=(0,),
                        start_index_map=(0,)),
                    (1,), mode=lax.GatherScatterMode.PROMISE_IN_BOUNDS)
                return col, lane

            def fire(i):
                col, _ = vat(i)
                pltpu.async_copy(tab.at[:, pl.ds(col, 128)],
                                 colbuf.at[i % K], sem)

            for i in range(K):
                fire(i)

            def item(i, _):
                pltpu.make_async_copy(
                    tab.at[:, pl.ds(0, 128)], colbuf.at[i % K], sem).wait()

                sv = jnp.full((16,), 0, jnp.int32) + (i % K)
                _, lane = vat(i)
                x0 = plsc.load_gather(colbuf, [sv, iota, lane])
                x1 = plsc.load_gather(colbuf, [sv, iota + 16, lane])
                ext_s[i, pl.ds(0, 16)] = x0
                ext_s[i, pl.ds(16, 16)] = x1

                @pl.when(i + K < _BPW)
                def _():
                    fire(i + K)

                return 0

            lax.fori_loop(0, _BPW, item, 0)
            pltpu.sync_copy(ext_s, out_h.at[pl.ds(wid * _BPW, _BPW), :])

        feature(sp_i_h, sp_tab_h, sp_out)
        feature(it_i_h, item_tab_h, true_out)
        feature(ng_i_h, item_tab_h, neg_out)

    return k(sp_i, it_i, ng_i, sp_tabT, item_tabT)


def _sc_hist(h_i, hist_tab):
    mesh = plsc.VectorSubcoreMesh(core_axis_name="c", subcore_axis_name="s")

    @functools.partial(
        pl.kernel,
        out_type=[jax.ShapeDtypeStruct((_B, _EMB), jnp.float32)],
        mesh=mesh,
        scratch_types=[
            pltpu.VMEM((_HIST, _BPW), jnp.int32),
            pltpu.VMEM((_HALF * _HIST, _EMB), jnp.float32),
            pltpu.VMEM((_BPW, _EMB), jnp.float32),
            pltpu.SemaphoreType.DMA,
        ],
        compiler_params=pltpu.CompilerParams(use_tc_tiling_on_sc=False, needs_layout_passes=False),
    )
    def k(h_i_h, hist_tab_h, hist_out, hidx, buf, pooled, sem):
        wid = lax.axis_index("s") * _NC + lax.axis_index("c")
        base = wid * _BPW

        pltpu.sync_copy(h_i_h.at[wid], hidx)

        inv = jnp.float32(1.0 / _HIST)
        for h in range(2):
            copies = [
                pltpu.async_copy(hist_tab_h.at[hidx.at[h * _CPH + c]],
                                 buf.at[pl.ds(c * _BPW, _BPW), :], sem)
                for c in range(_CPH)
            ]
            for cp in copies:
                cp.wait()

            def row_body(r, _):
                def inner(j, accs):
                    a0, a1 = accs
                    rb = r * _HIST + j * 5
                    for t in range(5):
                        a0 = a0 + buf[rb + t, pl.ds(0, 16)]
                        a1 = a1 + buf[rb + t, pl.ds(16, 16)]
                    return a0, a1

                z = jnp.zeros((16,), jnp.float32)
                a0, a1 = lax.fori_loop(0, _HIST // 5, inner, (z, z))
                rr = h * _HALF + r
                pooled[rr, pl.ds(0, 16)] = a0 * inv
                pooled[rr, pl.ds(16, 16)] = a1 * inv
                return 0

            lax.fori_loop(0, _HALF, row_body, 0)

        pltpu.sync_copy(pooled, hist_out.at[pl.ds(base, _BPW), :])

    return k(h_i, hist_tab)[0]


def _tc_loss(u_dense, sp_emb, hist_pooled, true_emb, neg_emb, W1, b1, W2, b2):
    def body(ud, sp, hp, te, ne, w1, b1r, w2, b2r, out):
        x = jnp.concatenate([ud[...], sp[...], hp[...]], axis=1)
        h = jnp.maximum(
            jnp.dot(x, w1[...], preferred_element_type=jnp.float32) + b1r[...], 0.0)
        ue = jnp.dot(h, w2[...], preferred_element_type=jnp.float32) + b2r[...]
        tl = jnp.sum(ue * te[...], axis=1, keepdims=True)
        nl = jnp.sum(ue * ne[...], axis=1, keepdims=True)
        d = nl - tl
        m = jnp.maximum(d, 0.0)
        out[...] = m + jnp.log(jnp.exp(-m) + jnp.exp(d - m))

    return pl.pallas_call(
        body,
        out_shape=jax.ShapeDtypeStruct((_B, 1), jnp.float32),
    )(u_dense, sp_emb, hist_pooled, true_emb, neg_emb,
      W1, b1.reshape(1, -1), W2, b2.reshape(1, -1))


def kernel(u_dense, u_sparse, u_hist, item_id, neg_ids,
           sparse_table, hist_table, item_table, W1, b1, W2, b2):
    sp_i = u_sparse.astype(jnp.int32).reshape(_NW, _BPW)
    it_i = item_id.astype(jnp.int32).reshape(_NW, _BPW)
    ng_i = neg_ids.astype(jnp.int32).reshape(_NW, _BPW)
    h_i = _remap(u_hist.astype(jnp.int32).reshape(_B * _HIST))
    h_i = h_i.reshape(_NW, _HIST, _BPW)
    sp_emb, true_emb, neg_emb = _sc_smalls(
        sp_i, it_i, ng_i, sparse_table.T, item_table.T)
    hist_tab = _pack_transpose(hist_table.T)
    h_i, _ = lax.optimization_barrier((h_i, sp_emb))
    hist_pooled = _sc_hist(h_i, hist_tab)
    loss = _tc_loss(u_dense, sp_emb, hist_pooled, true_emb, neg_emb, W1, b1, W2, b2)
    return loss.reshape(_B)
